# baseline (speedup 1.0000x reference)
import functools

import jax
import jax.numpy as jnp
from jax import lax
from jax.experimental import pallas as pl
from jax.experimental.pallas import tpu as pltpu, tpu_sc as plsc

NUM_COLS = 26
DIM = 32
CID = 4
VED = 28
BATCH = 16384
VOCAB1 = 100001

VPAD = 100352


CBLK = 12544
NCB = VPAD // CBLK


def _fat_body(t_ref, cid_ref, out_ref):
    i = pl.program_id(0)
    cidrow = cid_ref[pl.ds(i, 1), :]
    cidb = jnp.broadcast_to(cidrow[:, None, :], (CBLK // 4, 4, CID))
    t = jnp.transpose(t_ref[0])
    t4 = t.reshape(CBLK // 4, 4, VED)
    out_ref[...] = jnp.concatenate([cidb, t4], axis=2).reshape(CBLK // 4, 128)


def _build_fat(tab_t, col_ids):
    grid = (NUM_COLS, NCB)
    return pl.pallas_call(
        _fat_body,
        grid=grid,
        in_specs=[pl.BlockSpec((1, VED, CBLK), lambda i, c: (i, 0, c)),
                  pl.BlockSpec((NUM_COLS, CID), lambda i, c: (0, 0))],
        out_specs=pl.BlockSpec((CBLK // 4, 128),
                               lambda i, c: (i * NCB + c, 0)),
        out_shape=jax.ShapeDtypeStruct((NUM_COLS * VPAD // 4, 128),
                                       jnp.float32),
        compiler_params=pltpu.CompilerParams(
            dimension_semantics=("arbitrary", "arbitrary")),
    )(tab_t, col_ids)


NW = 32
CB = BATCH // NW
G = 128
NG = CB // G

_SC_PARAMS = pltpu.CompilerParams(
    use_tc_tiling_on_sc=False, needs_layout_passes=False)


def _sc_body(x_hbm, fat_hbm, out_hbm, ixs_v, ixg_v, vals_v, outT_v,
             semg0, semg1, semw):
    wid = lax.axis_index("s") * 2 + lax.axis_index("c")
    b0 = wid * CB
    i16 = lax.iota(jnp.int32, 16)

    pltpu.sync_copy(x_hbm.at[:, pl.ds(b0, CB)], ixs_v)

    def repack(i, sl):
        def rp(t, c2):
            x = ixs_v[i, pl.ds(t * 16, 16)]
            g = i * VPAD + x
            ixg_v[sl, t // (G // 16), pl.ds((t % (G // 16)) * 16, 16)] = g
            return c2
        lax.fori_loop(0, CB // 16, rp, 0)

    def fire(sl, sem):
        for k in range(NG):
            pltpu.async_copy(
                fat_hbm.at[ixg_v.at[sl, k]],
                vals_v.at[sl, pl.ds(k * G, G)], sem)

    def drain(sl, sem):
        for k in range(NG):
            pltpu.make_async_copy(
                fat_hbm.at[pl.ds(0, G)],
                vals_v.at[sl, pl.ds(k * G, G)], sem).wait()

    repack(0, 0)
    fire(0, semg0)

    def col_body(i, carry):
        sl = lax.rem(i, 2)

        @pl.when(i < NUM_COLS - 1)
        def _():
            repack(i + 1, 1 - sl)

        @pl.when((i < NUM_COLS - 1) & (sl == 0))
        def _():
            fire(1, semg1)

        @pl.when((i < NUM_COLS - 1) & (sl == 1))
        def _():
            fire(0, semg0)

        @pl.when(sl == 0)
        def _():
            drain(0, semg0)

        @pl.when(sl == 1)
        def _():
            drain(1, semg1)

        @pl.when(i >= 2)
        def _():
            pltpu.make_async_copy(
                out_hbm.at[0].at[:, pl.ds(0, CB)],
                outT_v.at[sl], semw).wait()

        for j in range(DIM):
            jv = jnp.full((16,), j, jnp.int32)

            def tr_k(k4, c2, _j=j, _jv=jv):
                for u in range(4):
                    o = (k4 * 4 + u) * 16
                    v = plsc.load_gather(vals_v, [jnp.full((16,), sl, jnp.int32),
                                                  o + i16, _jv])
                    outT_v[sl, _j, pl.ds(o, 16)] = v
                return c2
            lax.fori_loop(0, CB // 64, tr_k, 0)

        pltpu.async_copy(outT_v.at[sl], out_hbm.at[i].at[:, pl.ds(b0, CB)],
                         semw)
        return carry
    lax.fori_loop(0, NUM_COLS, col_body, 0)
    for _ in range(2):
        pltpu.make_async_copy(
            out_hbm.at[0].at[:, pl.ds(0, CB)], outT_v.at[0], semw).wait()


def kernel(x_categ, tables, col_ids):
    x_t = jnp.transpose(x_categ.astype(jnp.int32))
    tab_t = jnp.transpose(tables, (0, 2, 1))
    fat = _build_fat(tab_t, col_ids)
    fat32 = fat.reshape(NUM_COLS * VPAD, DIM)

    mesh = plsc.VectorSubcoreMesh(core_axis_name="c", subcore_axis_name="s")
    out_t = pl.kernel(
        _sc_body,
        out_type=jax.ShapeDtypeStruct((NUM_COLS, DIM, BATCH), jnp.float32),
        mesh=mesh,
        compiler_params=_SC_PARAMS,
        scratch_types=[
            pltpu.VMEM((NUM_COLS, CB), jnp.int32),
            pltpu.VMEM((2, NG, G), jnp.int32),
            pltpu.VMEM((2, CB, DIM), jnp.float32),
            pltpu.VMEM((2, DIM, CB), jnp.float32),
            pltpu.SemaphoreType.DMA,
            pltpu.SemaphoreType.DMA,
            pltpu.SemaphoreType.DMA,
        ],
    )(x_t, fat32)
    return jnp.transpose(out_t, (2, 0, 1))


# speedup vs baseline: 1.5161x; 1.5161x over previous
import functools

import jax
import jax.numpy as jnp
from jax import lax
from jax.experimental import pallas as pl
from jax.experimental.pallas import tpu as pltpu, tpu_sc as plsc

NUM_COLS = 26
DIM = 32
CID = 4
VED = 28
BATCH = 16384
VOCAB1 = 100001

V4 = 25088
VPAD = 4 * V4


CBLK = 12544
NCB = V4 // CBLK
NB2 = (VOCAB1 + CBLK - 1) // CBLK - 1


def _fat_body(t0, t1, t2, t3, cid_ref, out_ref):
    i = pl.program_id(0)
    cidrow = cid_ref[pl.ds(i, 1), :]
    cidb = jnp.broadcast_to(cidrow, (CBLK, CID))
    parts = []
    for t in (t0, t1, t2, t3):
        parts.append(cidb)
        parts.append(jnp.transpose(t[0]))
    out_ref[...] = jnp.concatenate(parts, axis=1)


def _build_fat(tab_t, col_ids):
    def in_spec(q):
        def imap(i, c):
            return (i, 0, jnp.minimum(q * NCB + c, NB2))
        return pl.BlockSpec((1, VED, CBLK), imap)

    grid = (NUM_COLS, NCB)
    return pl.pallas_call(
        _fat_body,
        grid=grid,
        in_specs=[in_spec(0), in_spec(1), in_spec(2), in_spec(3),
                  pl.BlockSpec((NUM_COLS, CID), lambda i, c: (0, 0))],
        out_specs=pl.BlockSpec((CBLK, 128), lambda i, c: (i * NCB + c, 0)),
        out_shape=jax.ShapeDtypeStruct((NUM_COLS * V4, 128), jnp.float32),
        compiler_params=pltpu.CompilerParams(
            dimension_semantics=("arbitrary", "arbitrary")),
    )(tab_t, tab_t, tab_t, tab_t, col_ids)


NW = 32
CB = BATCH // NW
G = 128
NG = CB // G

_SC_PARAMS = pltpu.CompilerParams(
    use_tc_tiling_on_sc=False, needs_layout_passes=False)


def _sc_body(x_hbm, fat_hbm, out_hbm, ixs_v, ixg_v, vals_v, outT_v,
             semg0, semg1, semw):
    wid = lax.axis_index("s") * 2 + lax.axis_index("c")
    b0 = wid * CB
    i16 = lax.iota(jnp.int32, 16)

    pltpu.sync_copy(x_hbm.at[:, pl.ds(b0, CB)], ixs_v)

    def repack(i, sl):
        def rp(t, c2):
            x = ixs_v[i, pl.ds(t * 16, 16)]
            g = i * VPAD + lax.rem(x, V4) * 4 + x // V4
            ixg_v[sl, t // (G // 16), pl.ds((t % (G // 16)) * 16, 16)] = g
            return c2
        lax.fori_loop(0, CB // 16, rp, 0)

    def fire(sl, sem):
        for k in range(NG):
            pltpu.async_copy(
                fat_hbm.at[ixg_v.at[sl, k]],
                vals_v.at[sl, pl.ds(k * G, G)], sem)

    def drain(sl, sem):
        for k in range(NG):
            pltpu.make_async_copy(
                fat_hbm.at[pl.ds(0, G)],
                vals_v.at[sl, pl.ds(k * G, G)], sem).wait()

    repack(0, 0)
    fire(0, semg0)

    def col_body(i, carry):
        sl = lax.rem(i, 2)

        @pl.when(i < NUM_COLS - 1)
        def _():
            repack(i + 1, 1 - sl)

        @pl.when((i < NUM_COLS - 1) & (sl == 0))
        def _():
            fire(1, semg1)

        @pl.when((i < NUM_COLS - 1) & (sl == 1))
        def _():
            fire(0, semg0)

        @pl.when(sl == 0)
        def _():
            drain(0, semg0)

        @pl.when(sl == 1)
        def _():
            drain(1, semg1)

        @pl.when(i >= 2)
        def _():
            pltpu.make_async_copy(
                out_hbm.at[0].at[:, pl.ds(0, CB)],
                outT_v.at[sl], semw).wait()

        slv = jnp.full((16,), sl, jnp.int32)
        for j in range(DIM):
            jv = jnp.full((16,), j, jnp.int32)
            for k in range(CB // 16):
                v = plsc.load_gather(vals_v, [slv, k * 16 + i16, jv])
                outT_v[sl, j, pl.ds(k * 16, 16)] = v

        pltpu.async_copy(outT_v.at[sl], out_hbm.at[i].at[:, pl.ds(b0, CB)],
                         semw)
        return carry
    lax.fori_loop(0, NUM_COLS, col_body, 0)
    for _ in range(2):
        pltpu.make_async_copy(
            out_hbm.at[0].at[:, pl.ds(0, CB)], outT_v.at[0], semw).wait()


def kernel(x_categ, tables, col_ids):
    x_t = jnp.transpose(x_categ.astype(jnp.int32))
    tab_t = jnp.transpose(tables, (0, 2, 1))
    fat = _build_fat(tab_t, col_ids)
    fat32 = fat.reshape(NUM_COLS * VPAD, DIM)

    mesh = plsc.VectorSubcoreMesh(core_axis_name="c", subcore_axis_name="s")
    out_t = pl.kernel(
        _sc_body,
        out_type=jax.ShapeDtypeStruct((NUM_COLS, DIM, BATCH), jnp.float32),
        mesh=mesh,
        compiler_params=_SC_PARAMS,
        scratch_types=[
            pltpu.VMEM((NUM_COLS, CB), jnp.int32),
            pltpu.VMEM((2, NG, G), jnp.int32),
            pltpu.VMEM((2, CB, DIM), jnp.float32),
            pltpu.VMEM((2, DIM, CB), jnp.float32),
            pltpu.SemaphoreType.DMA,
            pltpu.SemaphoreType.DMA,
            pltpu.SemaphoreType.DMA,
        ],
    )(x_t, fat32)
    return jnp.transpose(out_t, (2, 0, 1))
